# baseline (device time: 33592 ns/iter reference)
import jax
import jax.numpy as jnp
from jax import lax
from jax.experimental import pallas as pl
from jax.experimental.pallas import tpu as pltpu

N_DEV = 16


def kernel(x, router_W, route_idx, expert_W, shared_W):
    n_tok, d = x.shape
    h = shared_W.shape[1]
    n_local = expert_W.shape[0]
    chunk = n_tok // N_DEV

    def body(x_ref, rw_ref, idx_ref, ew_ref, sw_ref, out_ref,
             acc_ref, rs_buf, chunk_ref,
             rs_send, rs_recv, ag_send, ag_recv):
        my = lax.axis_index("i")

        barrier = pltpu.get_barrier_semaphore()
        for k in range(1, N_DEV):
            peer = lax.rem(my + k, N_DEV)
            pl.semaphore_signal(barrier, inc=1, device_id=(peer,),
                                device_id_type=pl.DeviceIdType.MESH)
        pl.semaphore_wait(barrier, N_DEV - 1)

        xv = x_ref[:, :]
        scores = jnp.dot(xv, rw_ref[:, :], preferred_element_type=jnp.float32)
        s_max = jnp.max(scores, axis=1, keepdims=True)
        denom = jnp.sum(jnp.exp(scores - s_max), axis=1, keepdims=True)
        p_top = 1.0 / denom
        idx = idx_ref[:, :]
        partial = jnp.zeros((n_tok, h), jnp.float32)
        for e in range(n_local):
            e_glob = my * n_local + e
            gate = jnp.where(idx == e_glob, p_top, 0.0)
            partial = partial + gate * jnp.dot(
                xv, ew_ref[e], preferred_element_type=jnp.float32)
        acc_ref[:, :, :] = partial.reshape(N_DEV, chunk, h)

        for k in range(1, N_DEV):
            peer = lax.rem(my + k, N_DEV)
            rdma = pltpu.make_async_remote_copy(
                src_ref=acc_ref.at[pl.ds(peer, 1)],
                dst_ref=rs_buf.at[pl.ds(k - 1, 1)],
                send_sem=rs_send.at[k - 1],
                recv_sem=rs_recv.at[k - 1],
                device_id=(peer,),
                device_id_type=pl.DeviceIdType.MESH,
            )
            rdma.start()

        for k in range(1, N_DEV):
            src_peer = lax.rem(my + N_DEV - k, N_DEV)
            recv = pltpu.make_async_remote_copy(
                src_ref=acc_ref.at[pl.ds(0, 1)],
                dst_ref=rs_buf.at[pl.ds(k - 1, 1)],
                send_sem=rs_send.at[k - 1],
                recv_sem=rs_recv.at[k - 1],
                device_id=(src_peer,),
                device_id_type=pl.DeviceIdType.MESH,
            )
            recv.wait_recv()

        own = acc_ref[pl.ds(my, 1), :, :].reshape(chunk, h)
        total = own + jnp.sum(rs_buf[:, :, :], axis=0)
        x_chunk = x_ref[pl.ds(my * chunk, chunk), :]
        total = total + jnp.dot(x_chunk, sw_ref[:, :],
                                preferred_element_type=jnp.float32)
        chunk_ref[:, :] = total
        out_ref[pl.ds(my * chunk, chunk), :] = total

        for k in range(1, N_DEV):
            peer = lax.rem(my + k, N_DEV)
            rdma = pltpu.make_async_remote_copy(
                src_ref=chunk_ref,
                dst_ref=out_ref.at[pl.ds(my * chunk, chunk), :],
                send_sem=ag_send.at[k - 1],
                recv_sem=ag_recv.at[k - 1],
                device_id=(peer,),
                device_id_type=pl.DeviceIdType.MESH,
            )
            rdma.start()

        for k in range(1, N_DEV):
            send = pltpu.make_async_remote_copy(
                src_ref=acc_ref.at[pl.ds(0, 1)],
                dst_ref=rs_buf.at[pl.ds(k - 1, 1)],
                send_sem=rs_send.at[k - 1],
                recv_sem=rs_recv.at[k - 1],
                device_id=(my,),
                device_id_type=pl.DeviceIdType.MESH,
            )
            send.wait_send()

        for k in range(1, N_DEV):
            src_peer = lax.rem(my + N_DEV - k, N_DEV)
            recv = pltpu.make_async_remote_copy(
                src_ref=chunk_ref,
                dst_ref=out_ref.at[pl.ds(src_peer * chunk, chunk), :],
                send_sem=ag_send.at[k - 1],
                recv_sem=ag_recv.at[k - 1],
                device_id=(src_peer,),
                device_id_type=pl.DeviceIdType.MESH,
            )
            recv.wait_recv()

        for k in range(1, N_DEV):
            send = pltpu.make_async_remote_copy(
                src_ref=chunk_ref,
                dst_ref=out_ref.at[pl.ds(0, chunk), :],
                send_sem=ag_send.at[k - 1],
                recv_sem=ag_recv.at[k - 1],
                device_id=(my,),
                device_id_type=pl.DeviceIdType.MESH,
            )
            send.wait_send()

    return pl.pallas_call(
        body,
        out_shape=jax.ShapeDtypeStruct((n_tok, h), jnp.float32),
        in_specs=[pl.BlockSpec(memory_space=pltpu.VMEM)] * 5,
        out_specs=pl.BlockSpec(memory_space=pltpu.VMEM),
        scratch_shapes=[
            pltpu.VMEM((N_DEV, chunk, h), jnp.float32),
            pltpu.VMEM((N_DEV - 1, chunk, h), jnp.float32),
            pltpu.VMEM((chunk, h), jnp.float32),
            pltpu.SemaphoreType.DMA((N_DEV - 1,)),
            pltpu.SemaphoreType.DMA((N_DEV - 1,)),
            pltpu.SemaphoreType.DMA((N_DEV - 1,)),
            pltpu.SemaphoreType.DMA((N_DEV - 1,)),
        ],
        compiler_params=pltpu.CompilerParams(collective_id=0),
    )(x, router_W, route_idx, expert_W, shared_W)


# device time: 25723 ns/iter; 1.3059x vs baseline; 1.3059x over previous
import jax
import jax.numpy as jnp
from jax import lax
from jax.experimental import pallas as pl
from jax.experimental.pallas import tpu as pltpu

N_DEV = 16


def kernel(x, router_W, route_idx, expert_W, shared_W):
    n_tok, d = x.shape
    h = shared_W.shape[1]
    n_local = expert_W.shape[0]
    chunk = n_tok // N_DEV

    def body(x_ref, rw_ref, idx_ref, ew_ref, sw_ref, out_ref,
             acc_ref, rs_buf, chunk_ref, ag_buf,
             rs_send, rs_recv, ag_send, ag_recv):
        my = lax.axis_index("i")

        barrier = pltpu.get_barrier_semaphore()
        for k in range(1, N_DEV):
            peer = lax.rem(my + k, N_DEV)
            pl.semaphore_signal(barrier, inc=1, device_id=(peer,),
                                device_id_type=pl.DeviceIdType.MESH)
        pl.semaphore_wait(barrier, N_DEV - 1)

        xv = x_ref[:, :]
        scores = jnp.dot(xv, rw_ref[:, :], preferred_element_type=jnp.float32)
        s_max = jnp.max(scores, axis=1, keepdims=True)
        denom = jnp.sum(jnp.exp(scores - s_max), axis=1, keepdims=True)
        p_top = 1.0 / denom
        idx = idx_ref[:, :]
        partial = jnp.zeros((n_tok, h), jnp.float32)
        for e in range(n_local):
            e_glob = my * n_local + e
            gate = jnp.where(idx == e_glob, p_top, 0.0)
            partial = partial + gate * jnp.dot(
                xv, ew_ref[e], preferred_element_type=jnp.float32)
        acc_ref[:, :, :] = partial.astype(jnp.bfloat16).reshape(N_DEV, chunk, h)

        for k in range(1, N_DEV):
            peer = lax.rem(my + k, N_DEV)
            rdma = pltpu.make_async_remote_copy(
                src_ref=acc_ref.at[pl.ds(peer, 1)],
                dst_ref=rs_buf.at[pl.ds(k - 1, 1)],
                send_sem=rs_send.at[k - 1],
                recv_sem=rs_recv.at[k - 1],
                device_id=(peer,),
                device_id_type=pl.DeviceIdType.MESH,
            )
            rdma.start()

        for k in range(1, N_DEV):
            src_peer = lax.rem(my + N_DEV - k, N_DEV)
            recv = pltpu.make_async_remote_copy(
                src_ref=acc_ref.at[pl.ds(0, 1)],
                dst_ref=rs_buf.at[pl.ds(k - 1, 1)],
                send_sem=rs_send.at[k - 1],
                recv_sem=rs_recv.at[k - 1],
                device_id=(src_peer,),
                device_id_type=pl.DeviceIdType.MESH,
            )
            recv.wait_recv()

        own = acc_ref[pl.ds(my, 1), :, :].astype(jnp.float32).reshape(chunk, h)
        total = own + jnp.sum(rs_buf[:, :, :].astype(jnp.float32), axis=0)
        x_chunk = x_ref[pl.ds(my * chunk, chunk), :]
        total = total + jnp.dot(x_chunk, sw_ref[:, :],
                                preferred_element_type=jnp.float32)
        total16 = total.astype(jnp.bfloat16).reshape(1, chunk, h)
        chunk_ref[:, :, :] = total16
        ag_buf[pl.ds(my, 1), :, :] = total16

        for k in range(1, N_DEV):
            peer = lax.rem(my + k, N_DEV)
            rdma = pltpu.make_async_remote_copy(
                src_ref=chunk_ref,
                dst_ref=ag_buf.at[pl.ds(my, 1)],
                send_sem=ag_send.at[k - 1],
                recv_sem=ag_recv.at[k - 1],
                device_id=(peer,),
                device_id_type=pl.DeviceIdType.MESH,
            )
            rdma.start()

        for k in range(1, N_DEV):
            send = pltpu.make_async_remote_copy(
                src_ref=acc_ref.at[pl.ds(0, 1)],
                dst_ref=rs_buf.at[pl.ds(k - 1, 1)],
                send_sem=rs_send.at[k - 1],
                recv_sem=rs_recv.at[k - 1],
                device_id=(my,),
                device_id_type=pl.DeviceIdType.MESH,
            )
            send.wait_send()

        for k in range(1, N_DEV):
            src_peer = lax.rem(my + N_DEV - k, N_DEV)
            recv = pltpu.make_async_remote_copy(
                src_ref=chunk_ref,
                dst_ref=ag_buf.at[pl.ds(src_peer, 1)],
                send_sem=ag_send.at[k - 1],
                recv_sem=ag_recv.at[k - 1],
                device_id=(src_peer,),
                device_id_type=pl.DeviceIdType.MESH,
            )
            recv.wait_recv()

        out_ref[:, :] = ag_buf[:, :, :].astype(jnp.float32).reshape(n_tok, h)

        for k in range(1, N_DEV):
            send = pltpu.make_async_remote_copy(
                src_ref=chunk_ref,
                dst_ref=ag_buf.at[pl.ds(0, 1)],
                send_sem=ag_send.at[k - 1],
                recv_sem=ag_recv.at[k - 1],
                device_id=(my,),
                device_id_type=pl.DeviceIdType.MESH,
            )
            send.wait_send()

    return pl.pallas_call(
        body,
        out_shape=jax.ShapeDtypeStruct((n_tok, h), jnp.float32),
        in_specs=[pl.BlockSpec(memory_space=pltpu.VMEM)] * 5,
        out_specs=pl.BlockSpec(memory_space=pltpu.VMEM),
        scratch_shapes=[
            pltpu.VMEM((N_DEV, chunk, h), jnp.bfloat16),
            pltpu.VMEM((N_DEV - 1, chunk, h), jnp.bfloat16),
            pltpu.VMEM((1, chunk, h), jnp.bfloat16),
            pltpu.VMEM((N_DEV, chunk, h), jnp.bfloat16),
            pltpu.SemaphoreType.DMA((N_DEV - 1,)),
            pltpu.SemaphoreType.DMA((N_DEV - 1,)),
            pltpu.SemaphoreType.DMA((N_DEV - 1,)),
            pltpu.SemaphoreType.DMA((N_DEV - 1,)),
        ],
        compiler_params=pltpu.CompilerParams(collective_id=0),
    )(x, router_W, route_idx, expert_W, shared_W)
